# baseline (speedup 1.0000x reference)
import jax
import jax.numpy as jnp
from jax import lax
from jax.experimental import pallas as pl
from jax.experimental.pallas import tpu as pltpu
from jax.experimental.pallas import tpu_sc as plsc

_ROWS = 100000
_DIM = 128
_K = 40000

_NC = 2
_NS = 16
_NW = _NC * _NS
_WROWS = _K // _NW
_CH = 250
_NCH = _WROWS // _CH
_NBUF = 3


def _sc_head_body(src_hbm, out_hbm, b0, b1, b2, si0, si1, si2, so0, so1, so2):
    bufs = [b0, b1, b2]
    sin = [si0, si1, si2]
    sout = [so0, so1, so2]
    wid = lax.axis_index("s") * _NC + lax.axis_index("c")
    base = wid * _WROWS

    in_cp = [None] * _NCH
    out_cp = [None] * _NCH
    for i in range(_NBUF - 1):
        in_cp[i] = pltpu.async_copy(
            src_hbm.at[pl.ds(base + i * _CH, _CH)], bufs[i % _NBUF], sin[i % _NBUF])
    for i in range(_NCH):
        j = i + _NBUF - 1
        if j < _NCH:
            if i >= 1:
                out_cp[i - 1].wait()
            in_cp[j] = pltpu.async_copy(
                src_hbm.at[pl.ds(base + j * _CH, _CH)], bufs[j % _NBUF], sin[j % _NBUF])
        in_cp[i].wait()
        out_cp[i] = pltpu.async_copy(
            bufs[i % _NBUF], out_hbm.at[pl.ds(base + i * _CH, _CH)], sout[i % _NBUF])
    for i in range(max(0, _NCH - _NBUF + 1), _NCH):
        out_cp[i].wait()


def _sc_head_copy(table):
    return pl.kernel(
        _sc_head_body,
        out_type=jax.ShapeDtypeStruct(table.shape, table.dtype),
        mesh=plsc.VectorSubcoreMesh(core_axis_name="c", subcore_axis_name="s"),
        scratch_types=(
            [pltpu.VMEM((_CH, _DIM), jnp.float32) for _ in range(_NBUF)]
            + [pltpu.SemaphoreType.DMA for _ in range(2 * _NBUF)]
        ),
        compiler_params=pltpu.CompilerParams(use_tc_tiling_on_sc=False),
    )(table)


def _copy_body(src_ref, out_ref):
    out_ref[...] = src_ref[...]


_SR_BLOCK = 25000


def _tc_full_copy(table):
    spec = pl.BlockSpec((_SR_BLOCK, _DIM), lambda i: (i, 0))
    return pl.pallas_call(
        _copy_body,
        grid=(_ROWS // _SR_BLOCK,),
        in_specs=[spec],
        out_specs=spec,
        out_shape=jax.ShapeDtypeStruct(table.shape, table.dtype),
    )(table)


_TAIL_BLOCK = 20000
_TAIL_OFF = _K // _TAIL_BLOCK


def _tail_body(p_ref, src_ref, out_ref):
    del p_ref
    out_ref[...] = src_ref[...]


def _tc_tail_fill(partial, table):
    spec = pl.BlockSpec((_TAIL_BLOCK, _DIM), lambda i: (i + _TAIL_OFF, 0))
    return pl.pallas_call(
        _tail_body,
        grid=((_ROWS - _K) // _TAIL_BLOCK,),
        in_specs=[pl.BlockSpec(memory_space=pl.ANY), spec],
        out_specs=spec,
        out_shape=jax.ShapeDtypeStruct(table.shape, table.dtype),
        input_output_aliases={0: 0},
    )(partial, table)


def kernel(sr_table, tg_table, rel_table, W, b):
    partial = _sc_head_copy(tg_table)
    sr_out = _tc_full_copy(sr_table)
    tg_out = _tc_tail_fill(partial, tg_table)
    return (sr_out, tg_out)


# speedup vs baseline: 1.2762x; 1.2762x over previous
import jax
import jax.numpy as jnp
from jax.experimental import pallas as pl

_ROWS = 100000
_BLOCK = 13336


def _copy2_body(sr_ref, tg_ref, sr_out, tg_out):
    sr_out[...] = sr_ref[...]
    tg_out[...] = tg_ref[...]


def kernel(sr_table, tg_table, rel_table, W, b):
    grid = (pl.cdiv(_ROWS, _BLOCK),)
    spec = pl.BlockSpec((_BLOCK, 128), lambda i: (i, 0))
    out = pl.pallas_call(
        _copy2_body,
        grid=grid,
        in_specs=[spec, spec],
        out_specs=[spec, spec],
        out_shape=[
            jax.ShapeDtypeStruct(sr_table.shape, sr_table.dtype),
            jax.ShapeDtypeStruct(tg_table.shape, tg_table.dtype),
        ],
    )(sr_table, tg_table)
    return (out[0], out[1])
